# Initial kernel scaffold; baseline (speedup 1.0000x reference)
import jax
import jax.numpy as jnp
from jax.experimental import pallas as pl


def kernel(ap_feat, ue_feat, edge_feat_ap2ue, edge_feat_ue2ap, params):
    raise NotImplementedError("write your pallas kernel here")


# speedup vs baseline: 2.1139x; 2.1139x over previous
import jax
import jax.numpy as jnp
from jax.experimental import pallas as pl
from jax.experimental.pallas import tpu as pltpu

N_AP = 128
N_UE = 512
D = 64
NE = N_AP * N_UE
TA = 8
TR = TA * N_UE
GRID = N_AP // TA
EPS = 1e-5
F32 = jnp.float32


def _dot(a, b):
    return jnp.dot(a, b, preferred_element_type=F32)


def _bn_coeffs(s, q, g, be):
    m = s / NE
    v = q / NE - m * m
    c = g * jax.lax.rsqrt(v + EPS)
    return c, be - c * m


def _chain3(e, pn3, aeT, b1, w2T, b2, w3T, b3):
    h = _dot(e, aeT).reshape(TA, N_UE, D)
    h = jnp.maximum(h + pn3 + b1.reshape(1, 1, D), 0.0).reshape(TR, D)
    h = jnp.maximum(_dot(h, w2T) + b2, 0.0)
    return jnp.maximum(_dot(h, w3T) + b3, 0.0)


def _node0_kernel(apf_ref, uef_ref, apw_ref, apv_ref, uew_ref, uev_ref,
                  anT_ref, aph_ref, ueh_ref, p1_ref, p3_ref, p5_ref, p6_ref):
    def one(x_ref, w_ref, v_ref):
        x = jnp.maximum(x_ref[...] * w_ref[...] + v_ref[0:1], 0.0)
        m = jnp.mean(x, axis=0, keepdims=True)
        v = jnp.mean((x - m) ** 2, axis=0, keepdims=True)
        return v_ref[1:2] * (x - m) * jax.lax.rsqrt(v + EPS) + v_ref[2:3]

    aph = one(apf_ref, apw_ref, apv_ref)
    ueh = one(uef_ref, uew_ref, uev_ref)
    aph_ref[...] = aph
    ueh_ref[...] = ueh
    anT = anT_ref[...]
    p1_ref[...] = _dot(ueh, anT[0])
    p3_ref[...] = _dot(aph, anT[1])
    p5_ref[...] = _dot(aph, anT[2])
    p6_ref[...] = _dot(ueh, anT[3])


def _prestats_kernel(ef12_ref, ef21_ref, prew_ref, prev_ref, st_ref):
    pb = prev_ref[0:1]
    prew = prew_ref[...]
    x12 = jnp.maximum(_dot(ef12_ref[...], prew) + pb, 0.0)
    x21 = jnp.maximum(_dot(ef21_ref[...], prew) + pb, 0.0)
    part = jnp.concatenate(
        [x12.sum(0, keepdims=True), (x12 * x12).sum(0, keepdims=True),
         x21.sum(0, keepdims=True), (x21 * x21).sum(0, keepdims=True)], axis=0)
    i = pl.program_id(0)

    @pl.when(i == 0)
    def _():
        st_ref[...] = part

    @pl.when(i > 0)
    def _():
        st_ref[...] += part


def _pass1_common(e12, e21, p1_ref, p3_ref, p5_ref, p6_ref,
                  aeT_ref, w2T_ref, w3T_ref, bias_ref,
                  x5_ref, x6_ref, s1_ref, s5_ref, s3_ref, s6_ref, st8_ref):
    aeT = aeT_ref[...]
    w2T = w2T_ref[...]
    w3T = w3T_ref[...]
    bias = bias_ref[...]
    p1b = p1_ref[...][None, :, :]
    p3b = p3_ref[...][:, None, :]
    p5b = p5_ref[...][:, None, :]
    p6b = p6_ref[...][None, :, :]

    def run(c, e, pn3):
        bc = bias[c]
        return _chain3(e, pn3, aeT[c], bc[0:1], w2T[c], bc[1:2], w3T[c],
                       bc[2:3])

    x1 = run(0, e21, p1b)
    x3 = run(1, e12, p3b)
    x5 = run(2, e21, p5b)
    x6 = run(3, e12, p6b)
    x5_ref[...] = x5
    x6_ref[...] = x6
    s1_ref[...] = jnp.sum(x1.reshape(TA, N_UE, D), axis=1)
    s5_ref[...] = jnp.sum(x5.reshape(TA, N_UE, D), axis=1)
    s3_part = jnp.sum(x3.reshape(TA, N_UE, D), axis=0)
    s6_part = jnp.sum(x6.reshape(TA, N_UE, D), axis=0)
    st_part = jnp.concatenate(
        [x1.sum(0, keepdims=True), (x1 * x1).sum(0, keepdims=True),
         x3.sum(0, keepdims=True), (x3 * x3).sum(0, keepdims=True),
         x5.sum(0, keepdims=True), (x5 * x5).sum(0, keepdims=True),
         x6.sum(0, keepdims=True), (x6 * x6).sum(0, keepdims=True)], axis=0)
    i = pl.program_id(0)

    @pl.when(i == 0)
    def _():
        s3_ref[...] = s3_part
        s6_ref[...] = s6_part
        st8_ref[...] = st_part

    @pl.when(i > 0)
    def _():
        s3_ref[...] += s3_part
        s6_ref[...] += s6_part
        st8_ref[...] += st_part


def _pass1_l1_kernel(ef12_ref, ef21_ref, prew_ref, prev_ref, prest_ref,
                     p1_ref, p3_ref, p5_ref, p6_ref,
                     aeT_ref, w2T_ref, w3T_ref, bias_ref,
                     x5_ref, x6_ref, s1_ref, s5_ref, s3_ref, s6_ref, st8_ref):
    prest = prest_ref[...]
    pb = prev_ref[0:1]
    pg = prev_ref[1:2]
    pbe = prev_ref[2:3]
    prew = prew_ref[...]
    c12, o12 = _bn_coeffs(prest[0:1], prest[1:2], pg, pbe)
    c21, o21 = _bn_coeffs(prest[2:3], prest[3:4], pg, pbe)
    e12 = c12 * jnp.maximum(_dot(ef12_ref[...], prew) + pb, 0.0) + o12
    e21 = c21 * jnp.maximum(_dot(ef21_ref[...], prew) + pb, 0.0) + o21
    _pass1_common(e12, e21, p1_ref, p3_ref, p5_ref, p6_ref,
                  aeT_ref, w2T_ref, w3T_ref, bias_ref,
                  x5_ref, x6_ref, s1_ref, s5_ref, s3_ref, s6_ref, st8_ref)


def _pass1_l2_kernel(x7_ref, bn7_ref, st7_ref,
                     p1_ref, p3_ref, p5_ref, p6_ref,
                     aeT_ref, w2T_ref, w3T_ref, bias_ref,
                     x5_ref, x6_ref, s1_ref, s5_ref, s3_ref, s6_ref, st8_ref):
    c7, o7 = _bn_coeffs(st7_ref[0:1], st7_ref[1:2], bn7_ref[0:1],
                        bn7_ref[1:2])
    e = c7 * x7_ref[...] + o7
    _pass1_common(e, e, p1_ref, p3_ref, p5_ref, p6_ref,
                  aeT_ref, w2T_ref, w3T_ref, bias_ref,
                  x5_ref, x6_ref, s1_ref, s5_ref, s3_ref, s6_ref, st8_ref)


def _node_kernel(aph_ref, ueh_ref, s1_ref, s3_ref, st8_ref, bn14_ref,
                 u2w_ref, u2b_ref, u2bn_ref, u4w_ref, u4b_ref, u4bn_ref,
                 anT_ref,
                 ap2_ref, ue2_ref, p1_ref, p3_ref, p5_ref, p6_ref):
    st8 = st8_ref[...]
    c1, o1 = _bn_coeffs(st8[0:1], st8[1:2], bn14_ref[0:1], bn14_ref[1:2])
    c3, o3 = _bn_coeffs(st8[2:3], st8[3:4], bn14_ref[2:3], bn14_ref[3:4])
    sum_r = c1 * s1_ref[...] + N_UE * o1
    sum_r2 = c3 * s3_ref[...] + N_AP * o3

    def node_chain(h0, s, w_ref, b_ref, bn_ref):
        w = w_ref[...]
        b = b_ref[...]
        h = jnp.maximum(_dot(h0, w[0]) + _dot(s, w[1]) + b[0:1], 0.0)
        h = jnp.maximum(_dot(h, w[2]) + b[1:2], 0.0)
        h = jnp.maximum(_dot(h, w[3]) + b[2:3], 0.0)
        m = jnp.mean(h, axis=0, keepdims=True)
        v = jnp.mean((h - m) ** 2, axis=0, keepdims=True)
        return bn_ref[0:1] * (h - m) * jax.lax.rsqrt(v + EPS) + bn_ref[1:2]

    ap2 = node_chain(aph_ref[...], sum_r, u2w_ref, u2b_ref, u2bn_ref)
    ue2 = node_chain(ueh_ref[...], sum_r2, u4w_ref, u4b_ref, u4bn_ref)
    ap2_ref[...] = ap2
    ue2_ref[...] = ue2
    anT = anT_ref[...]
    p1_ref[...] = _dot(ue2, anT[0])
    p3_ref[...] = _dot(ap2, anT[1])
    p5_ref[...] = _dot(ap2, anT[2])
    p6_ref[...] = _dot(ue2, anT[3])


def _pass2_common(e12, x5_ref, x6_ref, s5_ref, s6_ref, st8_ref, bn56_ref,
                  u7w_ref, u7b_ref, x7_ref, st2_ref):
    st8 = st8_ref[...]
    c5, o5 = _bn_coeffs(st8[4:5], st8[5:6], bn56_ref[0:1], bn56_ref[1:2])
    c6, o6 = _bn_coeffs(st8[6:7], st8[7:8], bn56_ref[2:3], bn56_ref[3:4])
    x5 = x5_ref[...].reshape(TA, N_UE, D)
    x6 = x6_ref[...].reshape(TA, N_UE, D)
    t1 = (c5.reshape(1, 1, D) * (s5_ref[...][:, None, :] - x5)
          + ((N_UE - 1) * o5).reshape(1, 1, D))
    t2 = (c6.reshape(1, 1, D) * (s6_ref[...][None, :, :] - x6)
          + ((N_AP - 1) * o6).reshape(1, 1, D))
    agg = (t1 + t2).reshape(TR, D)
    w = u7w_ref[...]
    b = u7b_ref[...]
    h = jnp.maximum(_dot(e12, w[0]) + _dot(agg, w[1]) + b[0:1], 0.0)
    h = jnp.maximum(_dot(h, w[2]) + b[1:2], 0.0)
    x7 = jnp.maximum(_dot(h, w[3]) + b[2:3], 0.0)
    x7_ref[...] = x7
    part = jnp.concatenate(
        [x7.sum(0, keepdims=True), (x7 * x7).sum(0, keepdims=True)], axis=0)
    i = pl.program_id(0)

    @pl.when(i == 0)
    def _():
        st2_ref[...] = part

    @pl.when(i > 0)
    def _():
        st2_ref[...] += part


def _pass2_l1_kernel(ef12_ref, prew_ref, prev_ref, prest_ref,
                     x5_ref, x6_ref, s5_ref, s6_ref, st8_ref, bn56_ref,
                     u7w_ref, u7b_ref, x7_ref, st2_ref):
    c12, o12 = _bn_coeffs(prest_ref[0:1], prest_ref[1:2], prev_ref[1:2],
                          prev_ref[2:3])
    e12 = c12 * jnp.maximum(_dot(ef12_ref[...], prew_ref[...])
                            + prev_ref[0:1], 0.0) + o12
    _pass2_common(e12, x5_ref, x6_ref, s5_ref, s6_ref, st8_ref, bn56_ref,
                  u7w_ref, u7b_ref, x7_ref, st2_ref)


def _pass2_l2_kernel(x7p_ref, bn7_ref, st7_ref,
                     x5_ref, x6_ref, s5_ref, s6_ref, st8_ref, bn56_ref,
                     u7w_ref, u7b_ref, x7_ref, st2_ref):
    c7, o7 = _bn_coeffs(st7_ref[0:1], st7_ref[1:2], bn7_ref[0:1],
                        bn7_ref[1:2])
    e12 = c7 * x7p_ref[...] + o7
    _pass2_common(e12, x5_ref, x6_ref, s5_ref, s6_ref, st8_ref, bn56_ref,
                  u7w_ref, u7b_ref, x7_ref, st2_ref)


def _post_kernel(x7_ref, bn7_ref, st7_ref, pw1T_ref, pb1_ref, pw2_ref,
                 out_ref):
    c7, o7 = _bn_coeffs(st7_ref[0:1], st7_ref[1:2], bn7_ref[0:1],
                        bn7_ref[1:2])
    e = c7 * x7_ref[...] + o7
    y = jnp.maximum(_dot(e, pw1T_ref[...]) + pb1_ref[...], 0.0)
    out_ref[...] = jax.nn.sigmoid(y * pw2_ref[...])


def _sds(shape):
    return jax.ShapeDtypeStruct(shape, F32)


def kernel(ap_feat, ue_feat, edge_feat_ap2ue, edge_feat_ue2ap, params):
    p = params
    u1, u3, u5, u6, u7 = p["u1"], p["u3"], p["u5"], p["u6"], p["u7"]

    aeT_s = jnp.stack([u1["w1"][:, :D].T, u3["w1"][:, :D].T,
                       u5["w1"][:, D:].T, u6["w1"][:, D:].T])
    anT_s = jnp.stack([u1["w1"][:, D:].T, u3["w1"][:, D:].T,
                       u5["w1"][:, :D].T, u6["w1"][:, :D].T])
    w2T_s = jnp.stack([c["w2"].T for c in (u1, u3, u5, u6)])
    w3T_s = jnp.stack([c["w3"].T for c in (u1, u3, u5, u6)])
    bias_s = jnp.stack([jnp.stack([c["b1"], c["b2"], c["b3"]])
                        for c in (u1, u3, u5, u6)])
    bn14 = jnp.stack([u1["g"], u1["be"], u3["g"], u3["be"]])
    bn56 = jnp.stack([u5["g"], u5["be"], u6["g"], u6["be"]])

    pe = p["pre_edge"]
    prew = pe["w1"].T
    prev = jnp.stack([pe["b1"], pe["g"], pe["be"]])

    def node_pack(c):
        w = jnp.stack([c["w1"][:, :D].T, c["w1"][:, D:].T,
                       c["w2"].T, c["w3"].T])
        b = jnp.stack([c["b1"], c["b2"], c["b3"]])
        bn = jnp.stack([c["g"], c["be"]])
        return w, b, bn

    u2w, u2b, u2bn = node_pack(p["u2"])
    u4w, u4b, u4bn = node_pack(p["u4"])

    bn7 = jnp.stack([u7["g"], u7["be"]])
    u7w = jnp.stack([u7["w1"][:, :D].T, u7["w1"][:, D:].T,
                     u7["w2"].T, u7["w3"].T])
    u7b = jnp.stack([u7["b1"], u7["b2"], u7["b3"]])

    pa, pu = p["pre_ap"], p["pre_ue"]
    apw = pa["w1"].T
    apv = jnp.stack([pa["b1"], pa["g"], pa["be"]])
    uew = pu["w1"].T
    uev = jnp.stack([pu["b1"], pu["g"], pu["be"]])

    pw1T = p["post"]["w1"].T
    pb1 = p["post"]["b1"].reshape(1, 1)
    pw2 = p["post"]["w2"]

    cp = pltpu.CompilerParams(dimension_semantics=("arbitrary",))
    grid = (GRID,)

    def espec():
        return pl.BlockSpec((TR, D), lambda i: (i, 0))

    def fspec():
        return pl.BlockSpec((TR, 2), lambda i: (i, 0))

    def aspec():
        return pl.BlockSpec((TA, D), lambda i: (i, 0))

    def full(shape):
        return pl.BlockSpec(shape, lambda i, _n=len(shape): (0,) * _n)

    aph, ueh, P1, P3, P5, P6 = pl.pallas_call(
        _node0_kernel,
        out_shape=[_sds((N_AP, D)), _sds((N_UE, D)), _sds((N_UE, D)),
                   _sds((N_AP, D)), _sds((N_AP, D)), _sds((N_UE, D))],
    )(ap_feat, ue_feat, apw, apv, uew, uev, anT_s)

    prest = pl.pallas_call(
        _prestats_kernel, grid=grid,
        in_specs=[fspec(), fspec(), full((2, D)), full((3, D))],
        out_specs=full((4, D)),
        out_shape=_sds((4, D)),
        compiler_params=cp,
    )(edge_feat_ap2ue, edge_feat_ue2ap, prew, prev)

    pass1_outs = dict(
        out_specs=[espec(), espec(), aspec(), aspec(),
                   full((N_UE, D)), full((N_UE, D)), full((8, D))],
        out_shape=[_sds((NE, D)), _sds((NE, D)), _sds((N_AP, D)),
                   _sds((N_AP, D)), _sds((N_UE, D)), _sds((N_UE, D)),
                   _sds((8, D))],
    )
    pspecs = [full((N_UE, D)), aspec(), aspec(), full((N_UE, D))]
    wspecs = [full((4, D, D)), full((4, D, D)), full((4, D, D)),
              full((4, 3, D))]

    x5, x6, S1, S5, S3, S6, st8 = pl.pallas_call(
        _pass1_l1_kernel, grid=grid,
        in_specs=[fspec(), fspec(), full((2, D)), full((3, D)),
                  full((4, D))] + pspecs + wspecs,
        compiler_params=cp, **pass1_outs,
    )(edge_feat_ap2ue, edge_feat_ue2ap, prew, prev, prest,
      P1, P3, P5, P6, aeT_s, w2T_s, w3T_s, bias_s)

    ap2, ue2, P1b, P3b, P5b, P6b = pl.pallas_call(
        _node_kernel,
        out_shape=[_sds((N_AP, D)), _sds((N_UE, D)), _sds((N_UE, D)),
                   _sds((N_AP, D)), _sds((N_AP, D)), _sds((N_UE, D))],
    )(aph, ueh, S1, S3, st8, bn14, u2w, u2b, u2bn, u4w, u4b, u4bn, anT_s)

    x7, st7 = pl.pallas_call(
        _pass2_l1_kernel, grid=grid,
        in_specs=[fspec(), full((2, D)), full((3, D)), full((4, D)),
                  espec(), espec(), aspec(), full((N_UE, D)),
                  full((8, D)), full((4, D)), full((4, D, D)),
                  full((3, D))],
        out_specs=[espec(), full((2, D))],
        out_shape=[_sds((NE, D)), _sds((2, D))],
        compiler_params=cp,
    )(edge_feat_ap2ue, prew, prev, prest, x5, x6, S5, S6, st8, bn56,
      u7w, u7b)

    x5b, x6b, S1b, S5b, S3b, S6b, st8b = pl.pallas_call(
        _pass1_l2_kernel, grid=grid,
        in_specs=[espec(), full((2, D)), full((2, D))] + pspecs + wspecs,
        compiler_params=cp, **pass1_outs,
    )(x7, bn7, st7, P1b, P3b, P5b, P6b, aeT_s, w2T_s, w3T_s, bias_s)

    x7b, st7b = pl.pallas_call(
        _pass2_l2_kernel, grid=grid,
        in_specs=[espec(), full((2, D)), full((2, D)),
                  espec(), espec(), aspec(), full((N_UE, D)),
                  full((8, D)), full((4, D)), full((4, D, D)),
                  full((3, D))],
        out_specs=[espec(), full((2, D))],
        out_shape=[_sds((NE, D)), _sds((2, D))],
        compiler_params=cp,
    )(x7, bn7, st7, x5b, x6b, S5b, S6b, st8b, bn56, u7w, u7b)

    out = pl.pallas_call(
        _post_kernel, grid=grid,
        in_specs=[espec(), full((2, D)), full((2, D)),
                  full((D, 1)), full((1, 1)), full((1, 1))],
        out_specs=pl.BlockSpec((TR, 1), lambda i: (i, 0)),
        out_shape=_sds((NE, 1)),
        compiler_params=cp,
    )(x7b, bn7, st7b, pw1T, pb1, pw2)
    return out


# speedup vs baseline: 2.2707x; 1.0742x over previous
import jax
import jax.numpy as jnp
from jax.experimental import pallas as pl
from jax.experimental.pallas import tpu as pltpu

N_AP = 128
N_UE = 512
D = 64
NE = N_AP * N_UE
TA = 8
TR = TA * N_UE
GRID = N_AP // TA
EPS = 1e-5
F32 = jnp.float32


def _dot(a, b):
    return jnp.dot(a, b, preferred_element_type=F32)


def _bn_coeffs(s, q, g, be):
    m = s / NE
    v = q / NE - m * m
    c = g * jax.lax.rsqrt(v + EPS)
    return c, be - c * m


def _chain3(e, pn3, aeT, b1, w2T, b2, w3T, b3):
    h = _dot(e, aeT).reshape(TA, N_UE, D)
    h = jnp.maximum(h + pn3 + b1.reshape(1, 1, D), 0.0).reshape(TR, D)
    h = jnp.maximum(_dot(h, w2T) + b2, 0.0)
    return jnp.maximum(_dot(h, w3T) + b3, 0.0)


def _node0_kernel(apf_ref, uef_ref, apw_ref, apv_ref, uew_ref, uev_ref,
                  anT_ref, aph_ref, ueh_ref, p1_ref, p3_ref, p5_ref, p6_ref):
    def one(x_ref, w_ref, v_ref):
        x = jnp.maximum(x_ref[...] * w_ref[...] + v_ref[0:1], 0.0)
        m = jnp.mean(x, axis=0, keepdims=True)
        v = jnp.mean((x - m) ** 2, axis=0, keepdims=True)
        return v_ref[1:2] * (x - m) * jax.lax.rsqrt(v + EPS) + v_ref[2:3]

    aph = one(apf_ref, apw_ref, apv_ref)
    ueh = one(uef_ref, uew_ref, uev_ref)
    aph_ref[...] = aph
    ueh_ref[...] = ueh
    anT = anT_ref[...]
    p1_ref[...] = _dot(ueh, anT[0])
    p3_ref[...] = _dot(aph, anT[1])
    p5_ref[...] = _dot(aph, anT[2])
    p6_ref[...] = _dot(ueh, anT[3])


def _prestats_kernel(ef12_ref, ef21_ref, prew_ref, prev_ref, st_ref):
    pb = prev_ref[0:1]
    prew = prew_ref[...]
    x12 = jnp.maximum(_dot(ef12_ref[...], prew) + pb, 0.0)
    x21 = jnp.maximum(_dot(ef21_ref[...], prew) + pb, 0.0)
    part = jnp.concatenate(
        [x12.sum(0, keepdims=True), (x12 * x12).sum(0, keepdims=True),
         x21.sum(0, keepdims=True), (x21 * x21).sum(0, keepdims=True)], axis=0)
    i = pl.program_id(0)

    @pl.when(i == 0)
    def _():
        st_ref[...] = part

    @pl.when(i > 0)
    def _():
        st_ref[...] += part


def _pass1_common(e12, e21, p1_ref, p3_ref, p5_ref, p6_ref,
                  aeT_ref, w2T_ref, w3T_ref, bias_ref,
                  x5_ref, x6_ref, s1_ref, s5_ref, s3_ref, s6_ref, st8_ref):
    aeT = aeT_ref[...]
    w2T = w2T_ref[...]
    w3T = w3T_ref[...]
    bias = bias_ref[...]
    p1b = p1_ref[...][None, :, :]
    p3b = p3_ref[...][:, None, :]
    p5b = p5_ref[...][:, None, :]
    p6b = p6_ref[...][None, :, :]

    def run(c, e, pn3):
        bc = bias[c]
        return _chain3(e, pn3, aeT[c], bc[0:1], w2T[c], bc[1:2], w3T[c],
                       bc[2:3])

    x1 = run(0, e21, p1b)
    x3 = run(1, e12, p3b)
    x5 = run(2, e21, p5b)
    x6 = run(3, e12, p6b)
    x5_ref[...] = x5
    x6_ref[...] = x6
    s1_ref[...] = jnp.sum(x1.reshape(TA, N_UE, D), axis=1)
    s5_ref[...] = jnp.sum(x5.reshape(TA, N_UE, D), axis=1)
    s3_part = jnp.sum(x3.reshape(TA, N_UE, D), axis=0)
    s6_part = jnp.sum(x6.reshape(TA, N_UE, D), axis=0)
    st_part = jnp.concatenate(
        [x1.sum(0, keepdims=True), (x1 * x1).sum(0, keepdims=True),
         x3.sum(0, keepdims=True), (x3 * x3).sum(0, keepdims=True),
         x5.sum(0, keepdims=True), (x5 * x5).sum(0, keepdims=True),
         x6.sum(0, keepdims=True), (x6 * x6).sum(0, keepdims=True)], axis=0)
    i = pl.program_id(0)

    @pl.when(i == 0)
    def _():
        s3_ref[...] = s3_part
        s6_ref[...] = s6_part
        st8_ref[...] = st_part

    @pl.when(i > 0)
    def _():
        s3_ref[...] += s3_part
        s6_ref[...] += s6_part
        st8_ref[...] += st_part


def _pass1_l1_kernel(ef12_ref, ef21_ref, prew_ref, prev_ref, prest_ref,
                     p1_ref, p3_ref, p5_ref, p6_ref,
                     aeT_ref, w2T_ref, w3T_ref, bias_ref,
                     x5_ref, x6_ref, s1_ref, s5_ref, s3_ref, s6_ref, st8_ref):
    prest = prest_ref[...]
    pb = prev_ref[0:1]
    pg = prev_ref[1:2]
    pbe = prev_ref[2:3]
    prew = prew_ref[...]
    c12, o12 = _bn_coeffs(prest[0:1], prest[1:2], pg, pbe)
    c21, o21 = _bn_coeffs(prest[2:3], prest[3:4], pg, pbe)
    e12 = c12 * jnp.maximum(_dot(ef12_ref[...], prew) + pb, 0.0) + o12
    e21 = c21 * jnp.maximum(_dot(ef21_ref[...], prew) + pb, 0.0) + o21
    _pass1_common(e12, e21, p1_ref, p3_ref, p5_ref, p6_ref,
                  aeT_ref, w2T_ref, w3T_ref, bias_ref,
                  x5_ref, x6_ref, s1_ref, s5_ref, s3_ref, s6_ref, st8_ref)


def _pass1_l2_kernel(x7_ref, bn7_ref, st7_ref, p5_ref, p6_ref,
                     aeT_ref, w2T_ref, w3T_ref, bias_ref,
                     x5_ref, x6_ref, s5_ref, s6_ref, st4_ref):
    c7, o7 = _bn_coeffs(st7_ref[0:1], st7_ref[1:2], bn7_ref[0:1],
                        bn7_ref[1:2])
    e = c7 * x7_ref[...] + o7
    aeT = aeT_ref[...]
    w2T = w2T_ref[...]
    w3T = w3T_ref[...]
    bias = bias_ref[...]
    x5 = _chain3(e, p5_ref[...][:, None, :], aeT[2], bias[2][0:1], w2T[2],
                 bias[2][1:2], w3T[2], bias[2][2:3])
    x6 = _chain3(e, p6_ref[...][None, :, :], aeT[3], bias[3][0:1], w2T[3],
                 bias[3][1:2], w3T[3], bias[3][2:3])
    x5_ref[...] = x5
    x6_ref[...] = x6
    s5_ref[...] = jnp.sum(x5.reshape(TA, N_UE, D), axis=1)
    s6_part = jnp.sum(x6.reshape(TA, N_UE, D), axis=0)
    st_part = jnp.concatenate(
        [x5.sum(0, keepdims=True), (x5 * x5).sum(0, keepdims=True),
         x6.sum(0, keepdims=True), (x6 * x6).sum(0, keepdims=True)], axis=0)
    i = pl.program_id(0)

    @pl.when(i == 0)
    def _():
        s6_ref[...] = s6_part
        st4_ref[...] = st_part

    @pl.when(i > 0)
    def _():
        s6_ref[...] += s6_part
        st4_ref[...] += st_part


def _node_kernel(aph_ref, ueh_ref, s1_ref, s3_ref, st8_ref, bn14_ref,
                 u2w_ref, u2b_ref, u2bn_ref, u4w_ref, u4b_ref, u4bn_ref,
                 anT_ref, p5_ref, p6_ref):
    st8 = st8_ref[...]
    c1, o1 = _bn_coeffs(st8[0:1], st8[1:2], bn14_ref[0:1], bn14_ref[1:2])
    c3, o3 = _bn_coeffs(st8[2:3], st8[3:4], bn14_ref[2:3], bn14_ref[3:4])
    sum_r = c1 * s1_ref[...] + N_UE * o1
    sum_r2 = c3 * s3_ref[...] + N_AP * o3

    def node_chain(h0, s, w_ref, b_ref, bn_ref):
        w = w_ref[...]
        b = b_ref[...]
        h = jnp.maximum(_dot(h0, w[0]) + _dot(s, w[1]) + b[0:1], 0.0)
        h = jnp.maximum(_dot(h, w[2]) + b[1:2], 0.0)
        h = jnp.maximum(_dot(h, w[3]) + b[2:3], 0.0)
        m = jnp.mean(h, axis=0, keepdims=True)
        v = jnp.mean((h - m) ** 2, axis=0, keepdims=True)
        return bn_ref[0:1] * (h - m) * jax.lax.rsqrt(v + EPS) + bn_ref[1:2]

    ap2 = node_chain(aph_ref[...], sum_r, u2w_ref, u2b_ref, u2bn_ref)
    ue2 = node_chain(ueh_ref[...], sum_r2, u4w_ref, u4b_ref, u4bn_ref)
    anT = anT_ref[...]
    p5_ref[...] = _dot(ap2, anT[2])
    p6_ref[...] = _dot(ue2, anT[3])


def _pass2_common(e12, x5_ref, x6_ref, s5_ref, s6_ref, st8_ref, bn56_ref,
                  u7w_ref, u7b_ref, x7_ref, st2_ref, st_off):
    st8 = st8_ref[...]
    c5, o5 = _bn_coeffs(st8[st_off:st_off + 1], st8[st_off + 1:st_off + 2],
                        bn56_ref[0:1], bn56_ref[1:2])
    c6, o6 = _bn_coeffs(st8[st_off + 2:st_off + 3],
                        st8[st_off + 3:st_off + 4],
                        bn56_ref[2:3], bn56_ref[3:4])
    x5 = x5_ref[...].reshape(TA, N_UE, D)
    x6 = x6_ref[...].reshape(TA, N_UE, D)
    t1 = (c5.reshape(1, 1, D) * (s5_ref[...][:, None, :] - x5)
          + ((N_UE - 1) * o5).reshape(1, 1, D))
    t2 = (c6.reshape(1, 1, D) * (s6_ref[...][None, :, :] - x6)
          + ((N_AP - 1) * o6).reshape(1, 1, D))
    agg = (t1 + t2).reshape(TR, D)
    w = u7w_ref[...]
    b = u7b_ref[...]
    h = jnp.maximum(_dot(e12, w[0]) + _dot(agg, w[1]) + b[0:1], 0.0)
    h = jnp.maximum(_dot(h, w[2]) + b[1:2], 0.0)
    x7 = jnp.maximum(_dot(h, w[3]) + b[2:3], 0.0)
    x7_ref[...] = x7
    part = jnp.concatenate(
        [x7.sum(0, keepdims=True), (x7 * x7).sum(0, keepdims=True)], axis=0)
    i = pl.program_id(0)

    @pl.when(i == 0)
    def _():
        st2_ref[...] = part

    @pl.when(i > 0)
    def _():
        st2_ref[...] += part


def _pass2_l1_kernel(ef12_ref, prew_ref, prev_ref, prest_ref,
                     x5_ref, x6_ref, s5_ref, s6_ref, st8_ref, bn56_ref,
                     u7w_ref, u7b_ref, x7_ref, st2_ref):
    c12, o12 = _bn_coeffs(prest_ref[0:1], prest_ref[1:2], prev_ref[1:2],
                          prev_ref[2:3])
    e12 = c12 * jnp.maximum(_dot(ef12_ref[...], prew_ref[...])
                            + prev_ref[0:1], 0.0) + o12
    _pass2_common(e12, x5_ref, x6_ref, s5_ref, s6_ref, st8_ref, bn56_ref,
                  u7w_ref, u7b_ref, x7_ref, st2_ref, st_off=4)


def _pass2_l2_kernel(x7p_ref, bn7_ref, st7_ref,
                     x5_ref, x6_ref, s5_ref, s6_ref, st8_ref, bn56_ref,
                     u7w_ref, u7b_ref, x7_ref, st2_ref):
    c7, o7 = _bn_coeffs(st7_ref[0:1], st7_ref[1:2], bn7_ref[0:1],
                        bn7_ref[1:2])
    e12 = c7 * x7p_ref[...] + o7
    _pass2_common(e12, x5_ref, x6_ref, s5_ref, s6_ref, st8_ref, bn56_ref,
                  u7w_ref, u7b_ref, x7_ref, st2_ref, st_off=0)


def _post_kernel(x7_ref, bn7_ref, st7_ref, pw1T_ref, pb1_ref, pw2_ref,
                 out_ref):
    c7, o7 = _bn_coeffs(st7_ref[0:1], st7_ref[1:2], bn7_ref[0:1],
                        bn7_ref[1:2])
    e = c7 * x7_ref[...] + o7
    y = jnp.maximum(_dot(e, pw1T_ref[...]) + pb1_ref[...], 0.0)
    out_ref[...] = jax.nn.sigmoid(y * pw2_ref[...])


def _sds(shape):
    return jax.ShapeDtypeStruct(shape, F32)


def kernel(ap_feat, ue_feat, edge_feat_ap2ue, edge_feat_ue2ap, params):
    p = params
    u1, u3, u5, u6, u7 = p["u1"], p["u3"], p["u5"], p["u6"], p["u7"]

    aeT_s = jnp.stack([u1["w1"][:, :D].T, u3["w1"][:, :D].T,
                       u5["w1"][:, D:].T, u6["w1"][:, D:].T])
    anT_s = jnp.stack([u1["w1"][:, D:].T, u3["w1"][:, D:].T,
                       u5["w1"][:, :D].T, u6["w1"][:, :D].T])
    w2T_s = jnp.stack([c["w2"].T for c in (u1, u3, u5, u6)])
    w3T_s = jnp.stack([c["w3"].T for c in (u1, u3, u5, u6)])
    bias_s = jnp.stack([jnp.stack([c["b1"], c["b2"], c["b3"]])
                        for c in (u1, u3, u5, u6)])
    bn14 = jnp.stack([u1["g"], u1["be"], u3["g"], u3["be"]])
    bn56 = jnp.stack([u5["g"], u5["be"], u6["g"], u6["be"]])

    pe = p["pre_edge"]
    prew = pe["w1"].T
    prev = jnp.stack([pe["b1"], pe["g"], pe["be"]])

    def node_pack(c):
        w = jnp.stack([c["w1"][:, :D].T, c["w1"][:, D:].T,
                       c["w2"].T, c["w3"].T])
        b = jnp.stack([c["b1"], c["b2"], c["b3"]])
        bn = jnp.stack([c["g"], c["be"]])
        return w, b, bn

    u2w, u2b, u2bn = node_pack(p["u2"])
    u4w, u4b, u4bn = node_pack(p["u4"])

    bn7 = jnp.stack([u7["g"], u7["be"]])
    u7w = jnp.stack([u7["w1"][:, :D].T, u7["w1"][:, D:].T,
                     u7["w2"].T, u7["w3"].T])
    u7b = jnp.stack([u7["b1"], u7["b2"], u7["b3"]])

    pa, pu = p["pre_ap"], p["pre_ue"]
    apw = pa["w1"].T
    apv = jnp.stack([pa["b1"], pa["g"], pa["be"]])
    uew = pu["w1"].T
    uev = jnp.stack([pu["b1"], pu["g"], pu["be"]])

    pw1T = p["post"]["w1"].T
    pb1 = p["post"]["b1"].reshape(1, 1)
    pw2 = p["post"]["w2"]

    cp = pltpu.CompilerParams(dimension_semantics=("arbitrary",))
    grid = (GRID,)

    def espec():
        return pl.BlockSpec((TR, D), lambda i: (i, 0))

    def fspec():
        return pl.BlockSpec((TR, 2), lambda i: (i, 0))

    def aspec():
        return pl.BlockSpec((TA, D), lambda i: (i, 0))

    def full(shape):
        return pl.BlockSpec(shape, lambda i, _n=len(shape): (0,) * _n)

    aph, ueh, P1, P3, P5, P6 = pl.pallas_call(
        _node0_kernel,
        out_shape=[_sds((N_AP, D)), _sds((N_UE, D)), _sds((N_UE, D)),
                   _sds((N_AP, D)), _sds((N_AP, D)), _sds((N_UE, D))],
    )(ap_feat, ue_feat, apw, apv, uew, uev, anT_s)

    prest = pl.pallas_call(
        _prestats_kernel, grid=grid,
        in_specs=[fspec(), fspec(), full((2, D)), full((3, D))],
        out_specs=full((4, D)),
        out_shape=_sds((4, D)),
        compiler_params=cp,
    )(edge_feat_ap2ue, edge_feat_ue2ap, prew, prev)

    pass1_outs = dict(
        out_specs=[espec(), espec(), aspec(), aspec(),
                   full((N_UE, D)), full((N_UE, D)), full((8, D))],
        out_shape=[_sds((NE, D)), _sds((NE, D)), _sds((N_AP, D)),
                   _sds((N_AP, D)), _sds((N_UE, D)), _sds((N_UE, D)),
                   _sds((8, D))],
    )
    pspecs = [full((N_UE, D)), aspec(), aspec(), full((N_UE, D))]
    wspecs = [full((4, D, D)), full((4, D, D)), full((4, D, D)),
              full((4, 3, D))]

    x5, x6, S1, S5, S3, S6, st8 = pl.pallas_call(
        _pass1_l1_kernel, grid=grid,
        in_specs=[fspec(), fspec(), full((2, D)), full((3, D)),
                  full((4, D))] + pspecs + wspecs,
        compiler_params=cp, **pass1_outs,
    )(edge_feat_ap2ue, edge_feat_ue2ap, prew, prev, prest,
      P1, P3, P5, P6, aeT_s, w2T_s, w3T_s, bias_s)

    P5b, P6b = pl.pallas_call(
        _node_kernel,
        out_shape=[_sds((N_AP, D)), _sds((N_UE, D))],
    )(aph, ueh, S1, S3, st8, bn14, u2w, u2b, u2bn, u4w, u4b, u4bn, anT_s)

    x7, st7 = pl.pallas_call(
        _pass2_l1_kernel, grid=grid,
        in_specs=[fspec(), full((2, D)), full((3, D)), full((4, D)),
                  espec(), espec(), aspec(), full((N_UE, D)),
                  full((8, D)), full((4, D)), full((4, D, D)),
                  full((3, D))],
        out_specs=[espec(), full((2, D))],
        out_shape=[_sds((NE, D)), _sds((2, D))],
        compiler_params=cp,
    )(edge_feat_ap2ue, prew, prev, prest, x5, x6, S5, S6, st8, bn56,
      u7w, u7b)

    x5b, x6b, S5b, S6b, st4b = pl.pallas_call(
        _pass1_l2_kernel, grid=grid,
        in_specs=[espec(), full((2, D)), full((2, D)), aspec(),
                  full((N_UE, D))] + wspecs,
        out_specs=[espec(), espec(), aspec(), full((N_UE, D)),
                   full((4, D))],
        out_shape=[_sds((NE, D)), _sds((NE, D)), _sds((N_AP, D)),
                   _sds((N_UE, D)), _sds((4, D))],
        compiler_params=cp,
    )(x7, bn7, st7, P5b, P6b, aeT_s, w2T_s, w3T_s, bias_s)

    x7b, st7b = pl.pallas_call(
        _pass2_l2_kernel, grid=grid,
        in_specs=[espec(), full((2, D)), full((2, D)),
                  espec(), espec(), aspec(), full((N_UE, D)),
                  full((4, D)), full((4, D)), full((4, D, D)),
                  full((3, D))],
        out_specs=[espec(), full((2, D))],
        out_shape=[_sds((NE, D)), _sds((2, D))],
        compiler_params=cp,
    )(x7, bn7, st7, x5b, x6b, S5b, S6b, st4b, bn56, u7w, u7b)

    out = pl.pallas_call(
        _post_kernel, grid=grid,
        in_specs=[espec(), full((2, D)), full((2, D)),
                  full((D, 1)), full((1, 1)), full((1, 1))],
        out_specs=pl.BlockSpec((TR, 1), lambda i: (i, 0)),
        out_shape=_sds((NE, 1)),
        compiler_params=cp,
    )(x7b, bn7, st7b, pw1T, pb1, pw2)
    return out
